# Initial kernel scaffold; baseline (speedup 1.0000x reference)
import jax
import jax.numpy as jnp
from jax.experimental import pallas as pl


def kernel(x, noise, adj_pos, params, t, edge_index):
    raise NotImplementedError("write your pallas kernel here")


# speedup vs baseline: 12.3975x; 12.3975x over previous
import functools

import jax
import jax.numpy as jnp
import numpy as np
from jax import lax
from jax.experimental import pallas as pl
from jax.experimental.pallas import tpu as pltpu
from jax.experimental.pallas import tpu_sc as plsc

N = 10000
E = 320000
D = 128
H = 256
T = 1000
P = 7
NHF = 2
NHU = 4

NC = 2
NS = 16
NW = NC * NS
M = 10240
SLAB = M // NW
RB = 512
TG = M // RB
CH = 80
TPAD = 1024

_MESH = dict(core_axis_name="c", subcore_axis_name="s", num_cores=NC,
             num_subcores=NS)

f32 = jnp.float32
i32 = jnp.int32


def _ln_rows(z):
    mu = jnp.mean(z, axis=-1, keepdims=True)
    va = jnp.mean((z - mu) ** 2, axis=-1, keepdims=True)
    return (z - mu) / jnp.sqrt(va + 1e-5)


def _lnstats_kernel(x_ref, xln_ref, st_ref):
    xl = _ln_rows(x_ref[...])
    xln_ref[...] = xl
    s0 = jnp.sum(xl, axis=0, keepdims=True)
    s1 = jnp.sum(xl * xl, axis=0, keepdims=True)
    upd = jnp.concatenate([s0, s1, jnp.zeros((6, xl.shape[-1]), f32)], axis=0)

    @pl.when(pl.program_id(0) == 0)
    def _():
        st_ref[...] = jnp.zeros_like(st_ref)

    st_ref[...] += upd


def _lnstats(x):
    return pl.pallas_call(
        _lnstats_kernel,
        grid=(TG,),
        in_specs=[pl.BlockSpec((RB, D), lambda i: (i, 0))],
        out_specs=[pl.BlockSpec((RB, D), lambda i: (i, 0)),
                   pl.BlockSpec((8, D), lambda i: (0, 0))],
        out_shape=[jax.ShapeDtypeStruct((M, D), f32),
                   jax.ShapeDtypeStruct((8, D), f32)],
    )(x)


def _prep_kernel(nz_ref, xln_ref, st_ref, o_ref):
    st = st_ref[...]
    miu = st[0:1, :] / N
    var = (st[1:2, :] - N * (miu * miu)) / (N - 1)
    std = jnp.sqrt(var)
    nzl = _ln_rows(nz_ref[...])
    nz = nzl * std + miu
    o_ref[...] = jnp.sign(xln_ref[...]) * jnp.abs(nz)


def _prep(noise, xln, st):
    return pl.pallas_call(
        _prep_kernel,
        grid=(TG,),
        in_specs=[pl.BlockSpec((RB, D), lambda i: (i, 0)),
                  pl.BlockSpec((RB, D), lambda i: (i, 0)),
                  pl.BlockSpec((8, D), lambda i: (0, 0))],
        out_specs=pl.BlockSpec((RB, D), lambda i: (i, 0)),
        out_shape=jax.ShapeDtypeStruct((M, D), f32),
    )(noise, xln, st)


def _gatproj_kernel(x_ref, w_ref, as_ref, ad_ref, hp_ref, ee_ref):
    z = jnp.dot(x_ref[...], w_ref[...], preferred_element_type=f32)
    hp_ref[0] = z[:, :128]
    hp_ref[1] = z[:, 128:]
    ee_ref[0] = jnp.dot(z, as_ref[...], preferred_element_type=f32)
    ee_ref[1] = jnp.dot(z, ad_ref[...], preferred_element_type=f32)


def _gatproj(h, W, As, Ad):
    k = h.shape[1]
    return pl.pallas_call(
        _gatproj_kernel,
        grid=(TG,),
        in_specs=[pl.BlockSpec((RB, k), lambda i: (i, 0)),
                  pl.BlockSpec((k, H), lambda i: (0, 0)),
                  pl.BlockSpec((H, 128), lambda i: (0, 0)),
                  pl.BlockSpec((H, 128), lambda i: (0, 0))],
        out_specs=[pl.BlockSpec((2, RB, 128), lambda i: (0, i, 0)),
                   pl.BlockSpec((2, RB, 128), lambda i: (0, i, 0))],
        out_shape=[jax.ShapeDtypeStruct((2, M, 128), f32),
                   jax.ShapeDtypeStruct((2, M, 128), f32)],
    )(h, W, As, Ad)


def _gatpost_kernel(n_ref, d_ref, ex_ref, *rest):
    if len(rest) == 2:
        prev_ref, o_ref = rest
    else:
        prev_ref, (o_ref,) = None, rest
    num = jnp.concatenate([n_ref[0], n_ref[1]], axis=1)
    den = d_ref[0][:, :16] + d_ref[1][:, :16]
    inv = 1.0 / (den + 1e-9)
    expd = jnp.dot(inv, ex_ref[...], preferred_element_type=f32)
    z = num * expd
    z = jnp.where(z > 0, z, jnp.exp(jnp.minimum(z, 0.0)) - 1.0)
    if prev_ref is not None:
        z = prev_ref[...] + z
    o_ref[...] = z


def _gatpost(num2, den2, ex, prev=None):
    ins = [num2, den2, ex] + ([prev] if prev is not None else [])
    in_specs = [pl.BlockSpec((2, RB, 128), lambda i: (0, i, 0)),
                pl.BlockSpec((2, RB, 128), lambda i: (0, i, 0)),
                pl.BlockSpec((16, H), lambda i: (0, 0))]
    if prev is not None:
        in_specs.append(pl.BlockSpec((RB, H), lambda i: (i, 0)))
    return pl.pallas_call(
        _gatpost_kernel,
        grid=(TG,),
        in_specs=in_specs,
        out_specs=pl.BlockSpec((RB, H), lambda i: (i, 0)),
        out_shape=jax.ShapeDtypeStruct((M, H), f32),
    )(*ins)


def _mmin_kernel(x_ref, w_ref, b_ref, te_ref, fe_ref, pe_ref, o_ref):
    z = jnp.dot(x_ref[...], w_ref[...], preferred_element_type=f32)
    o_ref[...] = z + b_ref[...] + te_ref[...] + fe_ref[...] + pe_ref[...]


def _mmin(xt, W, b, te, fe, pe):
    return pl.pallas_call(
        _mmin_kernel,
        grid=(TG,),
        in_specs=[pl.BlockSpec((RB, D), lambda i: (i, 0)),
                  pl.BlockSpec((D, H), lambda i: (0, 0)),
                  pl.BlockSpec((1, H), lambda i: (0, 0)),
                  pl.BlockSpec((RB, H), lambda i: (i, 0)),
                  pl.BlockSpec((RB, H), lambda i: (i, 0)),
                  pl.BlockSpec((RB, H), lambda i: (i, 0))],
        out_specs=pl.BlockSpec((RB, H), lambda i: (i, 0)),
        out_shape=jax.ShapeDtypeStruct((M, H), f32),
    )(xt, W, b[None], te, fe, pe)


def _posmlp_kernel(x_ref, w_ref, b_ref, a_ref, o_ref):
    z = jnp.dot(x_ref[...], w_ref[...], preferred_element_type=f32) + b_ref[...]
    z = _ln_rows(z)
    o_ref[...] = jnp.where(z >= 0, z, a_ref[...] * z)


def _posmlp(x, W, b, a):
    k = x.shape[1]
    a_row = jnp.full((1, H), a, f32)
    return pl.pallas_call(
        _posmlp_kernel,
        grid=(TG,),
        in_specs=[pl.BlockSpec((RB, k), lambda i: (i, 0)),
                  pl.BlockSpec((k, H), lambda i: (0, 0)),
                  pl.BlockSpec((1, H), lambda i: (0, 0)),
                  pl.BlockSpec((1, H), lambda i: (0, 0))],
        out_specs=pl.BlockSpec((RB, H), lambda i: (i, 0)),
        out_shape=jax.ShapeDtypeStruct((M, H), f32),
    )(x, W, b[None], a_row)


def _mmout_kernel(x_ref, w_ref, b_ref, xln_ref, o_ref):
    o_ref[0] = jnp.dot(x_ref[...], w_ref[...],
                       preferred_element_type=f32) + b_ref[...]
    o_ref[1] = xln_ref[...]


def _mmout(h, W, b, xln):
    return pl.pallas_call(
        _mmout_kernel,
        grid=(TG,),
        in_specs=[pl.BlockSpec((RB, H), lambda i: (i, 0)),
                  pl.BlockSpec((H, D), lambda i: (0, 0)),
                  pl.BlockSpec((1, D), lambda i: (0, 0)),
                  pl.BlockSpec((RB, D), lambda i: (i, 0))],
        out_specs=pl.BlockSpec((2, RB, D), lambda i: (0, i, 0)),
        out_shape=jax.ShapeDtypeStruct((2, M, D), f32),
    )(h, W, b[None], xln)


def _smoothdiv_kernel(s_ref, d_ref, o_ref, *, nh):
    deg = jnp.maximum(d_ref[0][:, nh:nh + 1] + d_ref[1][:, nh:nh + 1], 1.0)
    o_ref[0] = s_ref[0] / deg
    o_ref[1] = s_ref[1] / deg


def _smoothdiv(sm2, den2, nh):
    return pl.pallas_call(
        functools.partial(_smoothdiv_kernel, nh=nh),
        grid=(TG,),
        in_specs=[pl.BlockSpec((2, RB, D), lambda i: (0, i, 0)),
                  pl.BlockSpec((2, RB, 128), lambda i: (0, i, 0))],
        out_specs=pl.BlockSpec((2, RB, D), lambda i: (0, i, 0)),
        out_shape=jax.ShapeDtypeStruct((2, M, D), f32),
    )(sm2, den2)


def _loss_kernel(z_ref, o_ref, *, weight):
    xa = z_ref[0]
    ya = z_ref[1]
    sxy = jnp.sum(xa * ya, axis=1, keepdims=True)
    nx = jnp.sqrt(jnp.sum(xa * xa, axis=1, keepdims=True))
    ny = jnp.sqrt(jnp.sum(ya * ya, axis=1, keepdims=True))
    r = sxy / ((nx + 1e-8) * (ny + 1e-8))
    term = (1.0 - r) ** 2
    rows = lax.broadcasted_iota(i32, (RB, 1), 0) + pl.program_id(0) * RB
    term = jnp.where(rows < N, term, 0.0)
    s = jnp.sum(term) * (weight / N)
    lanes = lax.broadcasted_iota(i32, (1, 128), 1)

    @pl.when(pl.program_id(0) == 0)
    def _():
        o_ref[...] = jnp.zeros_like(o_ref)

    o_ref[...] += jnp.where(lanes == 0, s, 0.0)


def _loss(z2, weight):
    return pl.pallas_call(
        functools.partial(_loss_kernel, weight=weight),
        grid=(TG,),
        in_specs=[pl.BlockSpec((2, RB, D), lambda i: (0, i, 0))],
        out_specs=pl.BlockSpec((1, 128), lambda i: (0, 0)),
        out_shape=jax.ShapeDtypeStruct((1, 128), f32),
    )(z2)


def _zero_buf(buf, rows, cols):
    zv = jnp.zeros((16,), f32)

    def body(e, _):
        for c in range(cols // 16):
            buf[e, pl.ds(c * 16, 16)] = zv
        return 0

    lax.fori_loop(0, rows, body, 0)


def _sc_s1(t_pad, tt, abtab, xln, nz):
    def body(t_hbm, tt_hbm, ab_hbm, xln_hbm, nz_hbm,
             te_out, xt_out, t_v, ab_v, te_v, x_v, nz_v, xt_v):
        cid = lax.axis_index("c")
        sid = lax.axis_index("s")
        base = (cid * NS + sid) * SLAB

        def chunk(j, _):
            r0 = base + j * CH
            pltpu.sync_copy(t_hbm.at[pl.ds(r0, CH)], t_v)
            pltpu.sync_copy(tt_hbm.at[t_v], te_v)
            pltpu.sync_copy(te_v, te_out.at[pl.ds(r0, CH)])
            pltpu.sync_copy(ab_hbm.at[t_v], ab_v)
            pltpu.sync_copy(xln_hbm.at[pl.ds(r0, CH)], x_v)
            pltpu.sync_copy(nz_hbm.at[pl.ds(r0, CH)], nz_v)

            for e in range(CH):
                ab_row = ab_v[e, pl.ds(0, 16)]
                s = ab_row[0]
                m = ab_row[1]
                for c in range(D // 16):
                    sl = pl.ds(c * 16, 16)
                    xt_v[e, sl] = x_v[e, sl] * s + nz_v[e, sl] * m
            pltpu.sync_copy(xt_v, xt_out.at[pl.ds(r0, CH)])
            return 0

        lax.fori_loop(0, SLAB // CH, chunk, 0)

    return pl.kernel(
        body,
        out_type=[jax.ShapeDtypeStruct((M, H), f32),
                  jax.ShapeDtypeStruct((M, D), f32)],
        mesh=plsc.VectorSubcoreMesh(**_MESH),
        scratch_types=[
            pltpu.VMEM((CH,), i32),
            pltpu.VMEM((CH, 128), f32),
            pltpu.VMEM((CH, H), f32),
            pltpu.VMEM((CH, D), f32),
            pltpu.VMEM((CH, D), f32),
            pltpu.VMEM((CH, D), f32),
        ],
    )(t_pad, tt, abtab, xln, nz)


def _sc_s2(ee2, src, dst, nh):
    C2 = 40
    epw = E // NW
    nch = epw // C2

    def body(ee_hbm, src_hbm, dst_hbm, w_hbm, den_hbm,
             src_v, dst_v, rs_v, rd_v, wbuf, wpad, zbuf, den_s):
        cid = lax.axis_index("c")
        sid = lax.axis_index("s")
        wid = cid * NS + sid
        _zero_buf(zbuf, C2, 128)
        _zero_buf(wpad, C2, 128)
        lane = lax.iota(i32, 16)
        one16 = jnp.full((16,), 1.0, f32)
        zero16 = jnp.zeros((16,), f32)
        stripe = M // NS

        def zc(q, _):
            r0 = sid * stripe + q * C2
            pltpu.sync_copy(zbuf, den_s.at[pl.ds(r0, C2)])
            return 0

        lax.fori_loop(0, stripe // C2, zc, 0)
        plsc.subcore_barrier()

        def chunk(j, _):
            e0 = wid * epw + j * C2
            pltpu.sync_copy(src_hbm.at[pl.ds(e0, C2)], src_v)
            pltpu.sync_copy(dst_hbm.at[pl.ds(e0, C2)], dst_v)
            pltpu.sync_copy(ee_hbm.at[0].at[src_v], rs_v)
            pltpu.sync_copy(ee_hbm.at[1].at[dst_v], rd_v)
            for e in range(C2):
                ev = rs_v[e, pl.ds(0, 16)] + rd_v[e, pl.ds(0, 16)]
                ev = jnp.where(ev >= 0, ev, 0.2 * ev)
                wv = jnp.exp(ev)
                row = jnp.where(lane < nh, wv,
                                jnp.where(lane == nh, one16, zero16))
                wpad[e, pl.ds(0, 16)] = row
                wbuf[e, pl.ds(0, 16)] = row
            pltpu.sync_copy(wbuf, w_hbm.at[pl.ds(e0, C2)])
            pltpu.sync_copy(wpad, den_s.at[dst_v], add=True)
            return 0

        lax.fori_loop(0, nch, chunk, 0)
        plsc.subcore_barrier()

        def wc(q, _):
            r0 = sid * stripe + q * C2
            pltpu.sync_copy(den_s.at[pl.ds(r0, C2)],
                            den_hbm.at[cid].at[pl.ds(r0, C2)])
            return 0

        lax.fori_loop(0, stripe // C2, wc, 0)

    return pl.kernel(
        body,
        out_type=[jax.ShapeDtypeStruct((E, 16), f32),
                  jax.ShapeDtypeStruct((2, M, 128), f32)],
        mesh=plsc.VectorSubcoreMesh(**_MESH),
        scratch_types=[
            pltpu.VMEM((C2,), i32),
            pltpu.VMEM((C2,), i32),
            pltpu.VMEM((C2, 128), f32),
            pltpu.VMEM((C2, 128), f32),
            pltpu.VMEM((C2, 16), f32),
            pltpu.VMEM((C2, 128), f32),
            pltpu.VMEM((C2, 128), f32),
            pltpu.VMEM_SHARED((M, 128), f32),
        ],
    )(ee2, src, dst)


def _sc_s3(hp2, w, src, dst, nh):
    C3 = 40
    eps = E // NS
    nch = eps // C3
    hh = nh // 2
    hd = H // nh

    def body(hp_hbm, w_hbm, src_hbm, dst_hbm, num_hbm,
             src_a, src_b, dst_a, dst_b, wv_a, wv_b, rows_a, rows_b,
             sem_a, sem_b, zbuf, acc_s):
        cid = lax.axis_index("c")
        sid = lax.axis_index("s")
        srcs = (src_a, src_b)
        dsts = (dst_a, dst_b)
        wvs = (wv_a, wv_b)
        rows = (rows_a, rows_b)
        sems = (sem_a, sem_b)
        _zero_buf(zbuf, CH, 128)
        stripe = M // NS

        def zc(q, _):
            r0 = sid * stripe + q * CH
            pltpu.sync_copy(zbuf, acc_s.at[pl.ds(r0, CH)])
            return 0

        lax.fori_loop(0, stripe // CH, zc, 0)
        plsc.subcore_barrier()

        def gstart(j, b):
            e0 = sid * eps + j * C3
            pltpu.sync_copy(src_hbm.at[pl.ds(e0, C3)], srcs[b])
            pltpu.sync_copy(dst_hbm.at[pl.ds(e0, C3)], dsts[b])
            pltpu.sync_copy(w_hbm.at[pl.ds(e0, C3)], wvs[b])
            pltpu.make_async_copy(hp_hbm.at[cid].at[srcs[b]], rows[b],
                                  sems[b]).start()

        def consume(b):
            pltpu.make_async_copy(hp_hbm.at[cid].at[srcs[b]], rows[b],
                                  sems[b]).wait()

            def scale_block(base_head):
                for e in range(C3):
                    w_row = wvs[b][e, pl.ds(0, 16)]
                    for kh in range(hh):
                        s = w_row[base_head + kh]
                        for c in range(hd // 16):
                            sl = pl.ds(kh * hd + c * 16, 16)
                            rows[b][e, sl] = rows[b][e, sl] * s

            @pl.when(cid == 0)
            def _():
                scale_block(0)

            @pl.when(cid == 1)
            def _():
                scale_block(hh)
            pltpu.sync_copy(rows[b], acc_s.at[dsts[b]], add=True)

        gstart(0, 0)
        gstart(1, 1)

        def lp(j2, _):
            for b in range(2):
                consume(b)
                gstart(j2 * 2 + b + 2, b)
            return 0

        lax.fori_loop(0, nch // 2 - 1, lp, 0)
        consume(0)
        consume(1)
        plsc.subcore_barrier()

        def wc(q, _):
            r0 = sid * stripe + q * CH
            pltpu.sync_copy(acc_s.at[pl.ds(r0, CH)],
                            num_hbm.at[cid].at[pl.ds(r0, CH)])
            return 0

        lax.fori_loop(0, stripe // CH, wc, 0)

    return pl.kernel(
        body,
        out_type=jax.ShapeDtypeStruct((2, M, 128), f32),
        mesh=plsc.VectorSubcoreMesh(**_MESH),
        scratch_types=[
            pltpu.VMEM((C3,), i32),
            pltpu.VMEM((C3,), i32),
            pltpu.VMEM((C3,), i32),
            pltpu.VMEM((C3,), i32),
            pltpu.VMEM((C3, 16), f32),
            pltpu.VMEM((C3, 16), f32),
            pltpu.VMEM((C3, 128), f32),
            pltpu.VMEM((C3, 128), f32),
            pltpu.SemaphoreType.DMA,
            pltpu.SemaphoreType.DMA,
            pltpu.VMEM((CH, 128), f32),
            pltpu.VMEM_SHARED((M, 128), f32),
        ],
    )(hp2, w, src, dst)


def _sc_s4(z2, src, dst):
    eps = E // NS
    nch = eps // CH

    def body(z_hbm, src_hbm, dst_hbm, sm_hbm,
             src_a, src_b, dst_a, dst_b, rows_a, rows_b, sem_a, sem_b,
             zbuf, acc_s):
        cid = lax.axis_index("c")
        sid = lax.axis_index("s")
        srcs = (src_a, src_b)
        dsts = (dst_a, dst_b)
        rows = (rows_a, rows_b)
        sems = (sem_a, sem_b)
        _zero_buf(zbuf, CH, 128)
        stripe = M // NS

        def zc(q, _):
            r0 = sid * stripe + q * CH
            pltpu.sync_copy(zbuf, acc_s.at[pl.ds(r0, CH)])
            return 0

        lax.fori_loop(0, stripe // CH, zc, 0)
        plsc.subcore_barrier()

        def gstart(j, b):
            e0 = sid * eps + j * CH
            pltpu.sync_copy(src_hbm.at[pl.ds(e0, CH)], srcs[b])
            pltpu.sync_copy(dst_hbm.at[pl.ds(e0, CH)], dsts[b])
            pltpu.make_async_copy(z_hbm.at[cid].at[srcs[b]], rows[b],
                                  sems[b]).start()

        def consume(b):
            pltpu.make_async_copy(z_hbm.at[cid].at[srcs[b]], rows[b],
                                  sems[b]).wait()
            pltpu.sync_copy(rows[b], acc_s.at[dsts[b]], add=True)

        gstart(0, 0)
        gstart(1, 1)

        def lp(j2, _):
            for b in range(2):
                consume(b)
                gstart(j2 * 2 + b + 2, b)
            return 0

        lax.fori_loop(0, nch // 2 - 1, lp, 0)
        consume(0)
        consume(1)
        plsc.subcore_barrier()

        def wc(q, _):
            r0 = sid * stripe + q * CH
            pltpu.sync_copy(acc_s.at[pl.ds(r0, CH)],
                            sm_hbm.at[cid].at[pl.ds(r0, CH)])
            return 0

        lax.fori_loop(0, stripe // CH, wc, 0)

    return pl.kernel(
        body,
        out_type=jax.ShapeDtypeStruct((2, M, 128), f32),
        mesh=plsc.VectorSubcoreMesh(**_MESH),
        scratch_types=[
            pltpu.VMEM((CH,), i32),
            pltpu.VMEM((CH,), i32),
            pltpu.VMEM((CH,), i32),
            pltpu.VMEM((CH,), i32),
            pltpu.VMEM((CH, 128), f32),
            pltpu.VMEM((CH, 128), f32),
            pltpu.SemaphoreType.DMA,
            pltpu.SemaphoreType.DMA,
            pltpu.VMEM((CH, 128), f32),
            pltpu.VMEM_SHARED((M, 128), f32),
        ],
    )(z2, src, dst)


def _attn_mat(a, nh):
    hd = H // nh
    A = jnp.zeros((H, 128), f32)
    for h in range(nh):
        A = A.at[h * hd:(h + 1) * hd, h].set(a[h])
    return A


def _expand_mat(nh):
    hd = H // nh
    ex = np.zeros((16, H), np.float32)
    for h in range(nh):
        ex[h, h * hd:(h + 1) * hd] = 1.0
    return jnp.asarray(ex)


def _gat_layer(h, src, dst, W, a_s, a_d, nh, prev=None):
    hp2, ee2 = _gatproj(h, W, _attn_mat(a_s, nh), _attn_mat(a_d, nh))
    w, den2 = _sc_s2(ee2, src, dst, nh)
    num2 = _sc_s3(hp2, w, src, dst, nh)
    return _gatpost(num2, den2, _expand_mat(nh), prev), den2


def kernel(x, noise, adj_pos, params, t, edge_index):
    src, dst = edge_index[0], edge_index[1]
    betas = np.linspace(1e-4, 0.02, T, dtype=np.float64)
    ab = np.cumprod(1.0 - betas)
    abtab = np.zeros((TPAD, 128), np.float32)
    abtab[:T, 0] = np.sqrt(ab)
    abtab[:T, 1] = np.sqrt(1.0 - ab)
    abtab = jnp.asarray(abtab)

    x_pad = jnp.pad(x, ((0, M - N), (0, 0)))
    noise_pad = jnp.pad(noise, ((0, M - N), (0, 0)))
    t_pad = jnp.pad(t, (0, M - N))
    adj_pad = jnp.pad(adj_pos, ((0, M - N), (0, 128 - P)))
    tt_pad = jnp.pad(params['time_table'], ((0, TPAD - T), (0, 0)))

    xln, st = _lnstats(x_pad)
    nz = _prep(noise_pad, xln, st)
    te, xt = _sc_s1(t_pad, tt_pad, abtab, xln, nz)

    h = xln
    dens = None
    for l in range(2):
        h, dens = _gat_layer(h, src, dst, params['filt_W'][l],
                             params['filt_as'][l], params['filt_ad'][l], NHF)
    fe = h

    p = adj_pad
    p = _posmlp(p, jnp.pad(params['pos_W'][0], ((0, 128 - P), (0, 0))),
                params['pos_b'][0], params['pos_a'][0])
    p = _posmlp(p, params['pos_W'][1], params['pos_b'][1], params['pos_a'][1])
    pe = p

    h = _mmin(xt, params['in_W'], params['in_b'], te, fe, pe)
    for l in range(2):
        h, dens = _gat_layer(h, src, dst, params['unet_W'][l],
                             params['unet_as'][l], params['unet_ad'][l],
                             NHU, prev=h)
    z2 = _mmout(h, params['out_W'], params['out_b'], xln)

    l0 = _loss(z2, 1.0)
    sm2 = _sc_s4(z2, src, dst)
    z2 = _smoothdiv(sm2, dens, NHU)
    l1 = _loss(z2, 0.5)
    sm2 = _sc_s4(z2, src, dst)
    z2 = _smoothdiv(sm2, dens, NHU)
    l2 = _loss(z2, 0.3)
    return l0[0, 0] + l1[0, 0] + l2[0, 0]
